# Initial kernel scaffold; baseline (speedup 1.0000x reference)
import jax
import jax.numpy as jnp
from jax.experimental import pallas as pl


def kernel(length_q, length_k, embeddings_table):
    raise NotImplementedError("write your pallas kernel here")


# speedup vs baseline: 6.3519x; 6.3519x over previous
import jax
import jax.numpy as jnp
from jax import lax
from jax.experimental import pallas as pl
from jax.experimental.pallas import tpu as pltpu
from jax.experimental.pallas import tpu_sc as plsc

NUM_UNITS = 64
MAX_REL = 128
L = 2048
ROWS = 2 * MAX_REL + 1
TV_WORDS = ROWS * NUM_UNITS
REP_ROWS = 256
REP_WORDS = REP_ROWS * NUM_UNITS
E_PAD_ROWS = 4096
E_WORDS = E_PAD_ROWS * NUM_UNITS
ROW_WORDS = L * NUM_UNITS
HALF_WORDS = ROW_WORDS // 2
Q_PER_TILE = L // 32
WIN_WORDS = HALF_WORDS + (Q_PER_TILE - 1) * NUM_UNITS
OUT_WORDS = L * ROW_WORDS


def _sc_body(start_hbm, table_hbm, out_hbm, e_hbm,
             tv, rep, wbuf, sv, sem, bsem):
    c = lax.axis_index("c")
    s = lax.axis_index("s")

    pltpu.sync_copy(start_hbm, sv)
    start = sv[...][0]

    e_c = e_hbm.at[c]

    @pl.when(s == 0)
    def _build_e():
        pltpu.sync_copy(table_hbm, tv)
        for row, rng in ((0, range(8)), (2 * MAX_REL, range(8, 16))):
            base = row * NUM_UNITS
            v0 = tv[pl.ds(base, 16)]
            v1 = tv[pl.ds(base + 16, 16)]
            v2 = tv[pl.ds(base + 32, 16)]
            v3 = tv[pl.ds(base + 48, 16)]

            def body(r, _, v0=v0, v1=v1, v2=v2, v3=v3):
                o = r * NUM_UNITS
                rep[pl.ds(o, 16)] = v0
                rep[pl.ds(o + 16, 16)] = v1
                rep[pl.ds(o + 32, 16)] = v2
                rep[pl.ds(o + 48, 16)] = v3
                return 0

            lax.fori_loop(0, REP_ROWS, body, 0)
            for i in rng:
                pltpu.sync_copy(rep, e_c.at[pl.ds(i * REP_WORDS, REP_WORDS)])
        off = pl.multiple_of(start * NUM_UNITS, NUM_UNITS)
        pltpu.sync_copy(tv, e_c.at[pl.ds(off, TV_WORDS)])

    plsc.subcore_barrier()

    q_base = c * (L // 2) + s * Q_PER_TILE
    for half in range(2):
        a = (L - 1 - q_base - (Q_PER_TILE - 1)) * NUM_UNITS + half * HALF_WORDS
        pltpu.sync_copy(e_c.at[pl.ds(a, WIN_WORDS)], wbuf)
        for chunk in range(0, Q_PER_TILE, 16):
            cps = []
            for i in range(chunk, chunk + 16):
                q = q_base + i
                src = wbuf.at[pl.ds((Q_PER_TILE - 1 - i) * NUM_UNITS,
                                    HALF_WORDS)]
                dst = out_hbm.at[pl.ds(q * ROW_WORDS + half * HALF_WORDS,
                                       HALF_WORDS)]
                cps.append(pltpu.async_copy(src, dst, sem))
            for cp in cps:
                cp.wait()


def _make_sc_call():
    mesh = plsc.VectorSubcoreMesh(core_axis_name="c", subcore_axis_name="s")
    return pl.kernel(
        _sc_body,
        mesh=mesh,
        out_type=(
            jax.ShapeDtypeStruct((OUT_WORDS,), jnp.float32),
            jax.ShapeDtypeStruct((2, E_WORDS), jnp.float32),
        ),
        scratch_types=[
            pltpu.VMEM((TV_WORDS,), jnp.float32),
            pltpu.VMEM((REP_WORDS,), jnp.float32),
            pltpu.VMEM((WIN_WORDS,), jnp.float32),
            pltpu.VMEM((16,), jnp.int32),
            pltpu.SemaphoreType.DMA,
            pltpu.SemaphoreType.DMA,
        ],
        compiler_params=pltpu.CompilerParams(use_tc_tiling_on_sc=False),
    )


def kernel(length_q, length_k, embeddings_table):
    start = (L - 1) - MAX_REL + (length_k - length_q)
    start_arr = jnp.full((16,), start, jnp.int32)
    table_flat = embeddings_table.reshape(TV_WORDS)
    out_flat, _e = _make_sc_call()(start_arr, table_flat)
    return out_flat.reshape(L, L, NUM_UNITS)
